# Initial kernel scaffold; baseline (speedup 1.0000x reference)
import jax
import jax.numpy as jnp
from jax.experimental import pallas as pl


def kernel(x_agent, x_scene, src_as, dst_as, edge_attr_as, src_sa, dst_sa, edge_attr_sa, batch_agent, batch_scene, Wq, bq, Wk, bk, Wv, bv, We, Wskip, bskip, in_Wa, in_ba, in_Ws, in_bs, out_Wa, out_ba, out_Ws, out_bs, ln_g_a, ln_b_a, ln_g_s, ln_b_s):
    raise NotImplementedError("write your pallas kernel here")


# speedup vs baseline: 9.5017x; 9.5017x over previous
import functools
import jax
import jax.numpy as jnp
from jax.experimental import pallas as pl

Na, Ns, Din, E, De, Hd, L, H, B, Do = 50000, 10000, 128, 300000, 16, 128, 2, 4, 64, 128
Dh = Hd // H
ISQ = 1.0 / (float(Dh) ** 0.5)


def _silu(x):
    return x * jax.nn.sigmoid(x)


def _proj_body(x_ref, w_ref, b_ref, o_ref, *, act):
    y = jnp.dot(x_ref[...], w_ref[...], preferred_element_type=jnp.float32) + b_ref[...]
    if act == "silu":
        y = _silu(y)
    o_ref[...] = y


def _proj(x, w, b, act=None, blk=2048):
    n, d = x.shape
    k = w.shape[1]
    npad = pl.cdiv(n, blk) * blk
    if npad != n:
        x = jnp.pad(x, ((0, npad - n), (0, 0)))
    out = pl.pallas_call(
        functools.partial(_proj_body, act=act),
        grid=(npad // blk,),
        in_specs=[
            pl.BlockSpec((blk, d), lambda i: (i, 0)),
            pl.BlockSpec((d, k), lambda i: (0, 0)),
            pl.BlockSpec((k,), lambda i: (0,)),
        ],
        out_specs=pl.BlockSpec((blk, k), lambda i: (i, 0)),
        out_shape=jax.ShapeDtypeStruct((npad, k), jnp.float32),
    )(x, w, b)
    return out[:n]


def _ln_silu_body(x_ref, g_ref, b_ref, o_ref):
    x = x_ref[...]
    mu = jnp.mean(x, axis=-1, keepdims=True)
    var = jnp.mean((x - mu) * (x - mu), axis=-1, keepdims=True)
    y = (x - mu) * jax.lax.rsqrt(var + 1e-5) * g_ref[...] + b_ref[...]
    o_ref[...] = _silu(y)


def _ln_silu(x, g, b, blk=2048):
    n, d = x.shape
    npad = pl.cdiv(n, blk) * blk
    if npad != n:
        x = jnp.pad(x, ((0, npad - n), (0, 0)))
    out = pl.pallas_call(
        _ln_silu_body,
        grid=(npad // blk,),
        in_specs=[
            pl.BlockSpec((blk, d), lambda i: (i, 0)),
            pl.BlockSpec((d,), lambda i: (0,)),
            pl.BlockSpec((d,), lambda i: (0,)),
        ],
        out_specs=pl.BlockSpec((blk, d), lambda i: (i, 0)),
        out_shape=jax.ShapeDtypeStruct((npad, d), jnp.float32),
    )(x, g, b)
    return out[:n]


def _pool_body(x_ref, seg_ref, o_ref, *, nblocks, n):
    pi = pl.program_id(0)

    @pl.when(pi == 0)
    def _():
        o_ref[...] = jnp.zeros_like(o_ref)

    x = x_ref[...]
    blk = x.shape[0]
    rows = pi * blk + jax.lax.broadcasted_iota(jnp.int32, (blk, 1), 0)
    seg = seg_ref[...].reshape(blk, 1)
    onehot = jnp.where(
        (seg == jax.lax.broadcasted_iota(jnp.int32, (blk, B), 1)) & (rows < n),
        1.0, 0.0)
    ones = jnp.ones((blk, Do), jnp.float32)
    aug = jnp.concatenate([x, ones], axis=1)
    o_ref[...] += jnp.dot(onehot.T, aug, preferred_element_type=jnp.float32)

    @pl.when(pi == nblocks - 1)
    def _():
        acc = o_ref[...]
        cnt = jnp.clip(acc[:, Do:Do + 1], 1.0, None)
        o_ref[...] = acc / cnt


def _pool(x, seg, blk=2048):
    n, d = x.shape
    npad = pl.cdiv(n, blk) * blk
    if npad != n:
        x = jnp.pad(x, ((0, npad - n), (0, 0)))
        seg = jnp.pad(seg, (0, npad - n))
    nblocks = npad // blk
    out = pl.pallas_call(
        functools.partial(_pool_body, nblocks=nblocks, n=n),
        grid=(nblocks,),
        in_specs=[
            pl.BlockSpec((blk, d), lambda i: (i, 0)),
            pl.BlockSpec((blk,), lambda i: (i,)),
        ],
        out_specs=pl.BlockSpec((B, 2 * d), lambda i: (0, 0)),
        out_shape=jax.ShapeDtypeStruct((B, 2 * d), jnp.float32),
    )(x, seg)
    return out[:, :d]


def _edge_phase(q, kk, vv, ee, src, dst, n_dst):
    kj = kk[src].reshape(E, H, Dh) + ee.reshape(E, H, Dh)
    vj = vv[src].reshape(E, H, Dh) + ee.reshape(E, H, Dh)
    qd = q[dst].reshape(E, H, Dh)
    al = jnp.sum(qd * kj, axis=-1) * ISQ
    w = jnp.exp(al)
    msg = jnp.concatenate([(vj * w[:, :, None]).reshape(E, Hd), w], axis=1)
    acc = jax.ops.segment_sum(msg, dst, num_segments=n_dst)
    s = acc[:, Hd:].reshape(n_dst, H, 1)
    return (acc[:, :Hd].reshape(n_dst, H, Dh) / (s + 1e-16)).reshape(n_dst, Hd)


def _conv(x_src, x_dst, src, dst, ea, Wq, bq, Wk, bk, Wv, bv, We, Wskip, bskip):
    n_dst = x_dst.shape[0]
    q = _proj(x_dst, Wq, bq)
    kk = _proj(x_src, Wk, bk)
    vv = _proj(x_src, Wv, bv)
    ee = _proj(ea, We, jnp.zeros((Hd,), jnp.float32))
    skip = _proj(x_dst, Wskip, bskip)
    agg = _edge_phase(q, kk, vv, ee, src, dst, n_dst)
    return agg + skip


def kernel(x_agent, x_scene, src_as, dst_as, edge_attr_as, src_sa, dst_sa, edge_attr_sa, batch_agent, batch_scene, Wq, bq, Wk, bk, Wv, bv, We, Wskip, bskip, in_Wa, in_ba, in_Ws, in_bs, out_Wa, out_ba, out_Ws, out_bs, ln_g_a, ln_b_a, ln_g_s, ln_b_s):
    src_as = src_as.astype(jnp.int32)
    dst_as = dst_as.astype(jnp.int32)
    src_sa = src_sa.astype(jnp.int32)
    dst_sa = dst_sa.astype(jnp.int32)
    batch_agent = batch_agent.astype(jnp.int32)
    batch_scene = batch_scene.astype(jnp.int32)

    xa = _proj(x_agent, in_Wa, in_ba, act="silu")
    xs = _proj(x_scene, in_Ws, in_bs, act="silu")
    for l in range(L):
        ns = _conv(xa, xs, src_as, dst_as, edge_attr_as,
                   Wq[l, 0], bq[l, 0], Wk[l, 0], bk[l, 0], Wv[l, 0], bv[l, 0],
                   We[l, 0], Wskip[l, 0], bskip[l, 0])
        na = _conv(xs, xa, src_sa, dst_sa, edge_attr_sa,
                   Wq[l, 1], bq[l, 1], Wk[l, 1], bk[l, 1], Wv[l, 1], bv[l, 1],
                   We[l, 1], Wskip[l, 1], bskip[l, 1])
        xa = _ln_silu(na, ln_g_a, ln_b_a)
        xs = _ln_silu(ns, ln_g_s, ln_b_s)
    emb_a = _proj(xa, out_Wa, out_ba)
    emb_s = _proj(xs, out_Ws, out_bs)
    pa = _pool(emb_a, batch_agent)
    ps = _pool(emb_s, batch_scene)
    pc = jnp.concatenate([pa, ps], axis=-1)
    return (emb_a, emb_s, pa, ps, pc)


# speedup vs baseline: 9.6817x; 1.0189x over previous
import functools
import jax
import jax.numpy as jnp
from jax.experimental import pallas as pl

Na, Ns, Din, E, De, Hd, L, H, B, Do = 50000, 10000, 128, 300000, 16, 128, 2, 4, 64, 128
Dh = Hd // H
ISQ = 1.0 / (float(Dh) ** 0.5)


def _silu(x):
    return x * jax.nn.sigmoid(x)


def _proj_body(x_ref, w_ref, b_ref, o_ref, *, act):
    y = jnp.dot(x_ref[...], w_ref[...], preferred_element_type=jnp.float32) + b_ref[...]
    if act == "silu":
        y = _silu(y)
    o_ref[...] = y


def _proj(x, w, b, act=None, blk=2048):
    n, d = x.shape
    k = w.shape[1]
    npad = pl.cdiv(n, blk) * blk
    if npad != n:
        x = jnp.pad(x, ((0, npad - n), (0, 0)))
    out = pl.pallas_call(
        functools.partial(_proj_body, act=act),
        grid=(npad // blk,),
        in_specs=[
            pl.BlockSpec((blk, d), lambda i: (i, 0)),
            pl.BlockSpec((d, k), lambda i: (0, 0)),
            pl.BlockSpec((k,), lambda i: (0,)),
        ],
        out_specs=pl.BlockSpec((blk, k), lambda i: (i, 0)),
        out_shape=jax.ShapeDtypeStruct((npad, k), jnp.float32),
    )(x, w, b)
    return out[:n]


def _finish_body(acc_ref, skip_ref, g_ref, b_ref, o_ref):
    acc = acc_ref[...]
    s = acc[:, Hd:].reshape(-1, H, 1)
    v = acc[:, :Hd].reshape(-1, H, Dh)
    x = (v / (s + 1e-16)).reshape(-1, Hd) + skip_ref[...]
    mu = jnp.mean(x, axis=-1, keepdims=True)
    var = jnp.mean((x - mu) * (x - mu), axis=-1, keepdims=True)
    y = (x - mu) * jax.lax.rsqrt(var + 1e-5) * g_ref[...] + b_ref[...]
    o_ref[...] = _silu(y)


def _finish(acc, skip, g, b, blk=2048):
    n = acc.shape[0]
    npad = pl.cdiv(n, blk) * blk
    if npad != n:
        acc = jnp.pad(acc, ((0, npad - n), (0, 0)))
        skip = jnp.pad(skip, ((0, npad - n), (0, 0)))
    out = pl.pallas_call(
        _finish_body,
        grid=(npad // blk,),
        in_specs=[
            pl.BlockSpec((blk, Hd + H), lambda i: (i, 0)),
            pl.BlockSpec((blk, Hd), lambda i: (i, 0)),
            pl.BlockSpec((Hd,), lambda i: (0,)),
            pl.BlockSpec((Hd,), lambda i: (0,)),
        ],
        out_specs=pl.BlockSpec((blk, Hd), lambda i: (i, 0)),
        out_shape=jax.ShapeDtypeStruct((npad, Hd), jnp.float32),
    )(acc, skip, g, b)
    return out[:n]


def _pool_body(x_ref, seg_ref, o_ref, *, nblocks, n):
    pi = pl.program_id(0)

    @pl.when(pi == 0)
    def _():
        o_ref[...] = jnp.zeros_like(o_ref)

    x = x_ref[...]
    blk = x.shape[0]
    rows = pi * blk + jax.lax.broadcasted_iota(jnp.int32, (blk, 1), 0)
    seg = seg_ref[...].reshape(blk, 1)
    onehot = jnp.where(
        (seg == jax.lax.broadcasted_iota(jnp.int32, (blk, B), 1)) & (rows < n),
        1.0, 0.0)
    ones = jnp.ones((blk, Do), jnp.float32)
    aug = jnp.concatenate([x, ones], axis=1)
    o_ref[...] += jnp.dot(onehot.T, aug, preferred_element_type=jnp.float32)

    @pl.when(pi == nblocks - 1)
    def _():
        acc = o_ref[...]
        cnt = jnp.clip(acc[:, Do:Do + 1], 1.0, None)
        o_ref[...] = acc / cnt


def _pool(x, seg, blk=2048):
    n, d = x.shape
    npad = pl.cdiv(n, blk) * blk
    if npad != n:
        x = jnp.pad(x, ((0, npad - n), (0, 0)))
        seg = jnp.pad(seg, (0, npad - n))
    nblocks = npad // blk
    out = pl.pallas_call(
        functools.partial(_pool_body, nblocks=nblocks, n=n),
        grid=(nblocks,),
        in_specs=[
            pl.BlockSpec((blk, d), lambda i: (i, 0)),
            pl.BlockSpec((blk,), lambda i: (i,)),
        ],
        out_specs=pl.BlockSpec((B, 2 * d), lambda i: (0, 0)),
        out_shape=jax.ShapeDtypeStruct((B, 2 * d), jnp.float32),
    )(x, seg)
    return out[:, :d]


def _edge_phase(q, kk, vv, ee, src, dst, n_dst):
    kj = kk[src].reshape(E, H, Dh) + ee.reshape(E, H, Dh)
    vj = vv[src].reshape(E, H, Dh) + ee.reshape(E, H, Dh)
    qd = q[dst].reshape(E, H, Dh)
    al = jnp.sum(qd * kj, axis=-1) * ISQ
    w = jnp.exp(al)
    msg = jnp.concatenate([(vj * w[:, :, None]).reshape(E, Hd), w], axis=1)
    return jax.ops.segment_sum(msg, dst, num_segments=n_dst)


def _conv(x_src, x_dst, src, dst, ea, Wq, bq, Wk, bk, Wv, bv, We, Wskip,
          bskip, ln_g, ln_b):
    n_dst = x_dst.shape[0]
    q = _proj(x_dst, Wq, bq)
    kk = _proj(x_src, Wk, bk)
    vv = _proj(x_src, Wv, bv)
    ee = _proj(ea, We, jnp.zeros((Hd,), jnp.float32))
    skip = _proj(x_dst, Wskip, bskip)
    acc = _edge_phase(q, kk, vv, ee, src, dst, n_dst)
    return _finish(acc, skip, ln_g, ln_b)


def kernel(x_agent, x_scene, src_as, dst_as, edge_attr_as, src_sa, dst_sa, edge_attr_sa, batch_agent, batch_scene, Wq, bq, Wk, bk, Wv, bv, We, Wskip, bskip, in_Wa, in_ba, in_Ws, in_bs, out_Wa, out_ba, out_Ws, out_bs, ln_g_a, ln_b_a, ln_g_s, ln_b_s):
    src_as = src_as.astype(jnp.int32)
    dst_as = dst_as.astype(jnp.int32)
    src_sa = src_sa.astype(jnp.int32)
    dst_sa = dst_sa.astype(jnp.int32)
    batch_agent = batch_agent.astype(jnp.int32)
    batch_scene = batch_scene.astype(jnp.int32)

    xa = _proj(x_agent, in_Wa, in_ba, act="silu")
    xs = _proj(x_scene, in_Ws, in_bs, act="silu")
    for l in range(L):
        xs_new = _conv(xa, xs, src_as, dst_as, edge_attr_as,
                       Wq[l, 0], bq[l, 0], Wk[l, 0], bk[l, 0],
                       Wv[l, 0], bv[l, 0], We[l, 0], Wskip[l, 0], bskip[l, 0],
                       ln_g_s, ln_b_s)
        xa_new = _conv(xs, xa, src_sa, dst_sa, edge_attr_sa,
                       Wq[l, 1], bq[l, 1], Wk[l, 1], bk[l, 1],
                       Wv[l, 1], bv[l, 1], We[l, 1], Wskip[l, 1], bskip[l, 1],
                       ln_g_a, ln_b_a)
        xa, xs = xa_new, xs_new
    emb_a = _proj(xa, out_Wa, out_ba)
    emb_s = _proj(xs, out_Ws, out_bs)
    pa = _pool(emb_a, batch_agent)
    ps = _pool(emb_s, batch_scene)
    pc = jnp.concatenate([pa, ps], axis=-1)
    return (emb_a, emb_s, pa, ps, pc)
